# baseline (speedup 1.0000x reference)
import jax
import jax.numpy as jnp
from jax import lax
from jax.experimental import pallas as pl
from jax.experimental.pallas import tpu as pltpu
from jax.experimental.pallas import tpu_sc as plsc

NC = 2
NS = 16
NW = NC * NS
HALF = 8192
D = 128
B_W = HALF * 2 // NW


CH = 64
NCH = B_W // CH


def _copy_body(x_hbm, out_hbm, rows_v, gsem, ssem):
    wid = lax.axis_index("s") * NC + lax.axis_index("c")
    j = wid & 1
    i0 = (wid >> 1) * B_W
    gathers = [
        pltpu.async_copy(
            x_hbm.at[1 + j, pl.ds(i0 + c * CH, CH), 1], rows_v.at[c], gsem
        )
        for c in range(NCH)
    ]
    scatters = []
    for c in range(NCH):
        gathers[c].wait()
        scatters.append(
            pltpu.async_copy(
                rows_v.at[c], out_hbm.at[pl.ds(i0 + c * CH, CH), j], ssem
            )
        )
    for s in scatters:
        s.wait()


@jax.jit
def _run(xr):
    mesh = plsc.VectorSubcoreMesh(core_axis_name="c", subcore_axis_name="s")
    return pl.kernel(
        _copy_body,
        out_type=jax.ShapeDtypeStruct((HALF, 2, D), jnp.float32),
        mesh=mesh,
        scratch_types=[
            pltpu.VMEM((NCH, CH, D), jnp.float32),
            pltpu.SemaphoreType.DMA,
            pltpu.SemaphoreType.DMA,
        ],
    )(xr)


def kernel(x):
    xr = jnp.transpose(x, (1, 0, 2)).reshape(50, HALF, 2, D)
    return _run(xr)


# speedup vs baseline: 1.0121x; 1.0121x over previous
import jax
import jax.numpy as jnp
from jax import lax
from jax.experimental import pallas as pl
from jax.experimental.pallas import tpu as pltpu
from jax.experimental.pallas import tpu_sc as plsc

NC = 2
NS = 16
NW = NC * NS
HALF = 8192
D = 128
B_W = HALF * 2 // NW


CH = 256
NCH = B_W // CH


def _copy_body(x_hbm, out_hbm, rows_v, gsem, ssem):
    wid = lax.axis_index("s") * NC + lax.axis_index("c")
    j = wid & 1
    i0 = (wid >> 1) * B_W
    gathers = [
        pltpu.async_copy(
            x_hbm.at[1 + j, pl.ds(i0 + c * CH, CH), 1], rows_v.at[c], gsem
        )
        for c in range(NCH)
    ]
    scatters = []
    for c in range(NCH):
        gathers[c].wait()
        scatters.append(
            pltpu.async_copy(
                rows_v.at[c], out_hbm.at[pl.ds(i0 + c * CH, CH), j], ssem
            )
        )
    for s in scatters:
        s.wait()


@jax.jit
def _run(xr):
    mesh = plsc.VectorSubcoreMesh(core_axis_name="c", subcore_axis_name="s")
    return pl.kernel(
        _copy_body,
        out_type=jax.ShapeDtypeStruct((HALF, 2, D), jnp.float32),
        mesh=mesh,
        scratch_types=[
            pltpu.VMEM((NCH, CH, D), jnp.float32),
            pltpu.SemaphoreType.DMA,
            pltpu.SemaphoreType.DMA,
        ],
    )(xr)


def kernel(x):
    xr = jnp.transpose(x, (1, 0, 2)).reshape(50, HALF, 2, D)
    return _run(xr)


# speedup vs baseline: 1.0172x; 1.0051x over previous
import jax
import jax.numpy as jnp
from jax import lax
from jax.experimental import pallas as pl
from jax.experimental.pallas import tpu as pltpu
from jax.experimental.pallas import tpu_sc as plsc

NC = 2
NS = 16
NW = NC * NS
HALF = 8192
D = 128
B_I = HALF // NW
CH = 128
NCH = B_I // CH


def _copy_body(x_hbm, out_hbm, rows_v, gsem, ssem):
    wid = lax.axis_index("s") * NC + lax.axis_index("c")
    i0 = wid * B_I
    gathers = []
    for c in range(NCH):
        for j in range(2):
            gathers.append(
                pltpu.async_copy(
                    x_hbm.at[1 + j, pl.ds(i0 + c * CH, CH), 1],
                    rows_v.at[c, pl.ds(0, CH), j],
                    gsem,
                )
            )
    scatters = []
    for c in range(NCH):
        gathers[2 * c].wait()
        gathers[2 * c + 1].wait()
        scatters.append(
            pltpu.async_copy(
                rows_v.at[c], out_hbm.at[pl.ds(i0 + c * CH, CH)], ssem
            )
        )
    for s in scatters:
        s.wait()


@jax.jit
def _run(xr):
    mesh = plsc.VectorSubcoreMesh(core_axis_name="c", subcore_axis_name="s")
    return pl.kernel(
        _copy_body,
        out_type=jax.ShapeDtypeStruct((HALF, 2, D), jnp.float32),
        mesh=mesh,
        scratch_types=[
            pltpu.VMEM((NCH, CH, 2, D), jnp.float32),
            pltpu.SemaphoreType.DMA,
            pltpu.SemaphoreType.DMA,
        ],
    )(xr)


def kernel(x):
    xr = jnp.transpose(x, (1, 0, 2)).reshape(50, HALF, 2, D)
    return _run(xr)
